# baseline (speedup 1.0000x reference)
import functools

import jax
import jax.numpy as jnp
from jax import lax
from jax.experimental import pallas as pl
from jax.experimental.pallas import tpu as pltpu
from jax.experimental.pallas import tpu_sc as plsc

_NC = 2
_NS = 16
_NW = _NC * _NS

_V = 1000000
_B = 4096 * 200
_D = 64
_DP = 128
_CH = 128
_K = 4
_ROWS_G = _K * _CH
_PER_W = _B // _NW
_GROUPS = _PER_W // _ROWS_G

_NFULL = _V // _CH
_VREM = _V - _NFULL * _CH
_BLK_LO = _NFULL // _NW
_BLK_XTRA = _NFULL % _NW


def _tr_body(src, tailp, tbl, slab0, slab1, tr0, tr1,
             lsem0, lsem1, ssem0, ssem1):
  wid = lax.axis_index("s") * _NC + lax.axis_index("c")
  nblk = _BLK_LO + jnp.where(wid < _BLK_XTRA, 1, 0)
  viota = lax.broadcasted_iota(jnp.int32, (16,), 0)

  def blk(j):
    return (wid + _NW * j) * _CH

  def load(j, slab, sem):
    pltpu.async_copy(src.at[:, pl.ds(blk(j), _CH)], slab, sem)

  def wait_load(j, slab, sem):
    pltpu.make_async_copy(src.at[:, pl.ds(blk(j), _CH)], slab, sem).wait()

  rows16 = [viota + 16 * t for t in range(8)]
  rows_half = [(viota + 16 * t) // 2 for t in range(8)]
  half_off = [jnp.bitwise_and(viota + 16 * t, 1) * _D for t in range(8)]

  def transpose(slab, tr):
    @plsc.parallel_loop(0, _D, unroll=8)
    def _(j):
      for t in range(8):
        rv = rows16[t]
        dv = jnp.bitwise_and(rv + j, _D - 1)
        v = plsc.load_gather(slab, [dv, rv])
        plsc.store_scatter(tr, [rows_half[t], half_off[t] + dv], v)

  def blkh(j):
    return (wid + _NW * j) * (_CH // 2)

  def store(j, tr, sem):
    pltpu.async_copy(tr, tbl.at[pl.ds(blkh(j), _CH // 2)], sem)

  def wait_store(j, tr, sem):
    pltpu.make_async_copy(tr, tbl.at[pl.ds(blkh(j), _CH // 2)],
                          sem).wait()

  @pl.when(nblk > 0)
  def _():
    load(0, slab0, lsem0)

    def step(j, carry):
      even = lax.rem(j, 2) == 0

      @pl.when(jnp.logical_and(j + 1 < nblk, even))
      def _():
        load(j + 1, slab1, lsem1)

      @pl.when(jnp.logical_and(j + 1 < nblk, jnp.logical_not(even)))
      def _():
        load(j + 1, slab0, lsem0)

      @pl.when(even)
      def _():
        wait_load(j, slab0, lsem0)

        @pl.when(j >= 2)
        def _():
          wait_store(j - 2, tr0, ssem0)
        transpose(slab0, tr0)
        store(j, tr0, ssem0)

      @pl.when(jnp.logical_not(even))
      def _():
        wait_load(j, slab1, lsem1)

        @pl.when(j >= 2)
        def _():
          wait_store(j - 2, tr1, ssem1)
        transpose(slab1, tr1)
        store(j, tr1, ssem1)
      return carry

    lax.fori_loop(0, nblk, step, 0)
    last_even = lax.rem(nblk, 2) == 1

    @pl.when(jnp.logical_and(nblk >= 2, last_even))
    def _():
      wait_store(nblk - 2, tr1, ssem1)
      wait_store(nblk - 1, tr0, ssem0)

    @pl.when(jnp.logical_and(nblk >= 2, jnp.logical_not(last_even)))
    def _():
      wait_store(nblk - 2, tr0, ssem0)
      wait_store(nblk - 1, tr1, ssem1)

  @pl.when(wid == _NW - 1)
  def _():
    pltpu.sync_copy(tailp, tbl.at[pl.ds(_NFULL * _CH // 2, _VREM // 2)])


def _gather_body(xr, tbl, out, idx0, idx1, rows0, rows1,
                 isem0, isem1, gsem0, gsem1, ssem0, ssem1):
  wid = lax.axis_index("s") * _NC + lax.axis_index("c")
  base = wid * _PER_W

  def load_idx(g, dst, sem):
    return pltpu.async_copy(xr.at[pl.ds(base + g * _ROWS_G, _ROWS_G)],
                            dst, sem)

  def wait_idx(g, dst, sem):
    pltpu.make_async_copy(xr.at[pl.ds(base + g * _ROWS_G, _ROWS_G)],
                          dst, sem).wait()

  def gathers(idx_v, dst, sem):
    for k in range(_K):
      pltpu.async_copy(tbl.at[idx_v.at[pl.ds(k * _CH, _CH)]],
                       dst.at[pl.ds(k * _CH, _CH)], sem)

  def drain_g(idx_v, dst, sem):
    for k in range(_K):
      pltpu.make_async_copy(tbl.at[idx_v.at[pl.ds(k * _CH, _CH)]],
                            dst.at[pl.ds(k * _CH, _CH)], sem).wait()

  def store(g, src, sem):
    return pltpu.async_copy(
        src, out.at[pl.ds(base + g * _ROWS_G, _ROWS_G), pl.ds(0, _D)], sem)

  def drain_s(g, src, sem):
    pltpu.make_async_copy(
        src, out.at[pl.ds(base + g * _ROWS_G, _ROWS_G), pl.ds(0, _D)],
        sem).wait()

  load_idx(0, idx0, isem0)
  load_idx(1, idx1, isem1)
  wait_idx(0, idx0, isem0)
  gathers(idx0, rows0, gsem0)

  def step(p, carry):
    a = 2 * p
    b = a + 1

    wait_idx(b, idx1, isem1)

    @pl.when(p > 0)
    def _():
      drain_s(b - 2, rows1, ssem1)

    gathers(idx1, rows1, gsem1)

    drain_g(idx0, rows0, gsem0)

    @pl.when(p < _GROUPS // 2 - 1)
    def _():
      load_idx(a + 2, idx0, isem0)

    store(a, rows0, ssem0)
    drain_s(a, rows0, ssem0)

    @pl.when(p < _GROUPS // 2 - 1)
    def _():
      wait_idx(a + 2, idx0, isem0)
      gathers(idx0, rows0, gsem0)

    drain_g(idx1, rows1, gsem1)

    @pl.when(p < _GROUPS // 2 - 1)
    def _():
      load_idx(b + 2, idx1, isem1)

    store(b, rows1, ssem1)
    return carry

  lax.fori_loop(0, _GROUPS // 2, step, 0)
  drain_s(_GROUPS - 1, rows1, ssem1)


def _impl(x, embedding):
  xf = x.reshape(_B).astype(jnp.int32)
  tailp = embedding[_NFULL * _CH:].reshape(_VREM // 2, _DP)
  mesh = plsc.VectorSubcoreMesh(
      core_axis_name="c", subcore_axis_name="s",
      num_cores=_NC, num_subcores=_NS)

  tbl = pl.kernel(
      _tr_body,
      out_type=jax.ShapeDtypeStruct((_V // 2, _DP), jnp.float32),
      mesh=mesh,
      compiler_params=pltpu.CompilerParams(needs_layout_passes=False),
      scratch_types=[
          pltpu.VMEM((_D, _CH), jnp.float32),
          pltpu.VMEM((_D, _CH), jnp.float32),
          pltpu.VMEM((_CH // 2, _DP), jnp.float32),
          pltpu.VMEM((_CH // 2, _DP), jnp.float32),
          pltpu.SemaphoreType.DMA,
          pltpu.SemaphoreType.DMA,
          pltpu.SemaphoreType.DMA,
          pltpu.SemaphoreType.DMA,
      ],
  )(embedding.T, tailp)
  tbl = tbl.reshape(_V, _D)

  out = pl.kernel(
      _gather_body,
      out_type=jax.ShapeDtypeStruct((_B, _DP), jnp.float32),
      mesh=mesh,
      compiler_params=pltpu.CompilerParams(use_tc_tiling_on_sc=False),
      scratch_types=[
          pltpu.VMEM((_ROWS_G,), jnp.int32),
          pltpu.VMEM((_ROWS_G,), jnp.int32),
          pltpu.VMEM((_ROWS_G, _D), jnp.float32),
          pltpu.VMEM((_ROWS_G, _D), jnp.float32),
          pltpu.SemaphoreType.DMA,
          pltpu.SemaphoreType.DMA,
          pltpu.SemaphoreType.DMA,
          pltpu.SemaphoreType.DMA,
          pltpu.SemaphoreType.DMA,
          pltpu.SemaphoreType.DMA,
      ],
  )(xf, tbl)
  return out[:, :_D].reshape(4096, 200, _D)


kernel = jax.jit(_impl)


# speedup vs baseline: 1.0097x; 1.0097x over previous
import functools

import jax
import jax.numpy as jnp
from jax import lax
from jax.experimental import pallas as pl
from jax.experimental.pallas import tpu as pltpu
from jax.experimental.pallas import tpu_sc as plsc

_NC = 2
_NS = 16
_NW = _NC * _NS

_V = 1000000
_B = 4096 * 200
_D = 64
_DP = 128
_CH = 128
_K = 4
_ROWS_G = _K * _CH
_PER_W = _B // _NW
_GROUPS = _PER_W // _ROWS_G

_NFULL = _V // _CH
_VREM = _V - _NFULL * _CH
_BLK_LO = _NFULL // _NW
_BLK_XTRA = _NFULL % _NW


def _tr_body(src, tailp, tbl, slab0, slab1, tr0, tr1,
             lsem0, lsem1, ssem0, ssem1):
  wid = lax.axis_index("s") * _NC + lax.axis_index("c")
  nblk = _BLK_LO + jnp.where(wid < _BLK_XTRA, 1, 0)
  viota = lax.broadcasted_iota(jnp.int32, (16,), 0)

  def blk(j):
    return (wid + _NW * j) * _CH

  def load(j, slab, sem):
    pltpu.async_copy(src.at[:, pl.ds(blk(j), _CH)], slab, sem)

  def wait_load(j, slab, sem):
    pltpu.make_async_copy(src.at[:, pl.ds(blk(j), _CH)], slab, sem).wait()

  rows16 = [viota + 16 * t for t in range(8)]
  rows_half = [(viota + 16 * t) // 2 for t in range(8)]
  half_off = [jnp.bitwise_and(viota + 16 * t, 1) * _D for t in range(8)]

  def transpose(slab, tr):
    @plsc.parallel_loop(0, _D, unroll=4)
    def _(j):
      for t in range(8):
        rv = rows16[t]
        dv = jnp.bitwise_and(rv + j, _D - 1)
        v = plsc.load_gather(slab, [dv, rv])
        plsc.store_scatter(tr, [rows_half[t], half_off[t] + dv], v)

  def blkh(j):
    return (wid + _NW * j) * (_CH // 2)

  def store(j, tr, sem):
    pltpu.async_copy(tr, tbl.at[pl.ds(blkh(j), _CH // 2)], sem)

  def wait_store(j, tr, sem):
    pltpu.make_async_copy(tr, tbl.at[pl.ds(blkh(j), _CH // 2)],
                          sem).wait()

  @pl.when(nblk > 0)
  def _():
    load(0, slab0, lsem0)

    def step(j, carry):
      even = lax.rem(j, 2) == 0

      @pl.when(jnp.logical_and(j + 1 < nblk, even))
      def _():
        load(j + 1, slab1, lsem1)

      @pl.when(jnp.logical_and(j + 1 < nblk, jnp.logical_not(even)))
      def _():
        load(j + 1, slab0, lsem0)

      @pl.when(even)
      def _():
        wait_load(j, slab0, lsem0)

        @pl.when(j >= 2)
        def _():
          wait_store(j - 2, tr0, ssem0)
        transpose(slab0, tr0)
        store(j, tr0, ssem0)

      @pl.when(jnp.logical_not(even))
      def _():
        wait_load(j, slab1, lsem1)

        @pl.when(j >= 2)
        def _():
          wait_store(j - 2, tr1, ssem1)
        transpose(slab1, tr1)
        store(j, tr1, ssem1)
      return carry

    lax.fori_loop(0, nblk, step, 0)
    last_even = lax.rem(nblk, 2) == 1

    @pl.when(jnp.logical_and(nblk >= 2, last_even))
    def _():
      wait_store(nblk - 2, tr1, ssem1)
      wait_store(nblk - 1, tr0, ssem0)

    @pl.when(jnp.logical_and(nblk >= 2, jnp.logical_not(last_even)))
    def _():
      wait_store(nblk - 2, tr0, ssem0)
      wait_store(nblk - 1, tr1, ssem1)

  @pl.when(wid == _NW - 1)
  def _():
    pltpu.sync_copy(tailp, tbl.at[pl.ds(_NFULL * _CH // 2, _VREM // 2)])


def _gather_body(xr, tbl, out, idx0, idx1, rows0, rows1,
                 isem0, isem1, gsem0, gsem1, ssem0, ssem1):
  wid = lax.axis_index("s") * _NC + lax.axis_index("c")
  base = wid * _PER_W

  def load_idx(g, dst, sem):
    return pltpu.async_copy(xr.at[pl.ds(base + g * _ROWS_G, _ROWS_G)],
                            dst, sem)

  def wait_idx(g, dst, sem):
    pltpu.make_async_copy(xr.at[pl.ds(base + g * _ROWS_G, _ROWS_G)],
                          dst, sem).wait()

  def gathers(idx_v, dst, sem):
    for k in range(_K):
      pltpu.async_copy(tbl.at[idx_v.at[pl.ds(k * _CH, _CH)]],
                       dst.at[pl.ds(k * _CH, _CH)], sem)

  def drain_g(idx_v, dst, sem):
    for k in range(_K):
      pltpu.make_async_copy(tbl.at[idx_v.at[pl.ds(k * _CH, _CH)]],
                            dst.at[pl.ds(k * _CH, _CH)], sem).wait()

  def store(g, src, sem):
    return pltpu.async_copy(
        src, out.at[pl.ds(base + g * _ROWS_G, _ROWS_G), pl.ds(0, _D)], sem)

  def drain_s(g, src, sem):
    pltpu.make_async_copy(
        src, out.at[pl.ds(base + g * _ROWS_G, _ROWS_G), pl.ds(0, _D)],
        sem).wait()

  load_idx(0, idx0, isem0)
  load_idx(1, idx1, isem1)
  wait_idx(0, idx0, isem0)
  gathers(idx0, rows0, gsem0)

  def step(p, carry):
    a = 2 * p
    b = a + 1

    wait_idx(b, idx1, isem1)

    @pl.when(p > 0)
    def _():
      drain_s(b - 2, rows1, ssem1)

    gathers(idx1, rows1, gsem1)

    drain_g(idx0, rows0, gsem0)

    @pl.when(p < _GROUPS // 2 - 1)
    def _():
      load_idx(a + 2, idx0, isem0)

    store(a, rows0, ssem0)
    drain_s(a, rows0, ssem0)

    @pl.when(p < _GROUPS // 2 - 1)
    def _():
      wait_idx(a + 2, idx0, isem0)
      gathers(idx0, rows0, gsem0)

    drain_g(idx1, rows1, gsem1)

    @pl.when(p < _GROUPS // 2 - 1)
    def _():
      load_idx(b + 2, idx1, isem1)

    store(b, rows1, ssem1)
    return carry

  lax.fori_loop(0, _GROUPS // 2, step, 0)
  drain_s(_GROUPS - 1, rows1, ssem1)


def _impl(x, embedding):
  xf = x.reshape(_B).astype(jnp.int32)
  tailp = embedding[_NFULL * _CH:].reshape(_VREM // 2, _DP)
  mesh = plsc.VectorSubcoreMesh(
      core_axis_name="c", subcore_axis_name="s",
      num_cores=_NC, num_subcores=_NS)

  tbl = pl.kernel(
      _tr_body,
      out_type=jax.ShapeDtypeStruct((_V // 2, _DP), jnp.float32),
      mesh=mesh,
      compiler_params=pltpu.CompilerParams(needs_layout_passes=False),
      scratch_types=[
          pltpu.VMEM((_D, _CH), jnp.float32),
          pltpu.VMEM((_D, _CH), jnp.float32),
          pltpu.VMEM((_CH // 2, _DP), jnp.float32),
          pltpu.VMEM((_CH // 2, _DP), jnp.float32),
          pltpu.SemaphoreType.DMA,
          pltpu.SemaphoreType.DMA,
          pltpu.SemaphoreType.DMA,
          pltpu.SemaphoreType.DMA,
      ],
  )(embedding.T, tailp)
  tbl = tbl.reshape(_V, _D)

  out = pl.kernel(
      _gather_body,
      out_type=jax.ShapeDtypeStruct((_B, _DP), jnp.float32),
      mesh=mesh,
      compiler_params=pltpu.CompilerParams(use_tc_tiling_on_sc=False),
      scratch_types=[
          pltpu.VMEM((_ROWS_G,), jnp.int32),
          pltpu.VMEM((_ROWS_G,), jnp.int32),
          pltpu.VMEM((_ROWS_G, _D), jnp.float32),
          pltpu.VMEM((_ROWS_G, _D), jnp.float32),
          pltpu.SemaphoreType.DMA,
          pltpu.SemaphoreType.DMA,
          pltpu.SemaphoreType.DMA,
          pltpu.SemaphoreType.DMA,
          pltpu.SemaphoreType.DMA,
          pltpu.SemaphoreType.DMA,
      ],
  )(xf, tbl)
  return out[:, :_D].reshape(4096, 200, _D)


kernel = jax.jit(_impl)
